# baseline (speedup 1.0000x reference)
import functools

import jax
import jax.numpy as jnp
from jax import lax
from jax.experimental import pallas as pl
from jax.experimental.pallas import tpu as pltpu
from jax.experimental.pallas import tpu_sc as plsc

D = 64
NC = 2
NS = 16
NW = NC * NS
SUB = 128
NSUB = 4
C = SUB * NSUB


def _gather_body(word_hbm, xt_hbm, out_hbm, idx_v, rows0, rows1, sem_g,
                 sem_w0, sem_w1):
    n_rows = out_hbm.shape[0]
    per_w = n_rows // NW
    chunks = per_w // C
    idx_rows_per_w = per_w // SUB
    wid = lax.axis_index("c") * NS + lax.axis_index("s")
    rows = (rows0, rows1)
    sem_w = (sem_w0, sem_w1)

    def pair(t, carry):
        ixrow = pl.multiple_of(wid * idx_rows_per_w + t * 2 * NSUB, 8)
        pltpu.sync_copy(xt_hbm.at[pl.ds(ixrow, 2 * NSUB)], idx_v)
        for b in range(2):
            k = 2 * t + b
            @pl.when(t >= 1)
            def _():
                pltpu.make_async_copy(
                    rows[b], out_hbm.at[pl.ds(0, C)], sem_w[b]).wait()

            cps = [
                pltpu.async_copy(word_hbm.at[idx_v.at[b * NSUB + j]],
                                 rows[b].at[pl.ds(j * SUB, SUB)], sem_g)
                for j in range(NSUB)
            ]
            for cp in cps:
                cp.wait()
            row0 = pl.multiple_of(wid * per_w + k * C, C)
            pltpu.async_copy(rows[b], out_hbm.at[pl.ds(row0, C)], sem_w[b])
        return carry

    lax.fori_loop(0, chunks // 2, pair, 0)
    for b in range(2):
        pltpu.make_async_copy(rows[b], out_hbm.at[pl.ds(0, C)],
                              sem_w[b]).wait()


@functools.partial(jax.jit, static_argnames=("n_rows",))
def _gather(word_table, xt2d, n_rows):
    mesh = plsc.VectorSubcoreMesh(core_axis_name="c", subcore_axis_name="s",
                                  num_cores=NC, num_subcores=NS)
    return pl.kernel(
        _gather_body,
        out_type=jax.ShapeDtypeStruct((n_rows, D), jnp.float32),
        mesh=mesh,
        compiler_params=pltpu.CompilerParams(use_tc_tiling_on_sc=False),
        scratch_types=[
            pltpu.VMEM((2 * NSUB, SUB), jnp.int32),
            pltpu.VMEM((C, D), jnp.float32),
            pltpu.VMEM((C, D), jnp.float32),
            pltpu.SemaphoreType.DMA,
            pltpu.SemaphoreType.DMA,
            pltpu.SemaphoreType.DMA,
        ],
    )(word_table, xt2d)


def _addpos_body(w_ref, pos_ref, o_ref):
    l = pl.program_id(0)
    w2 = w_ref[0]
    eye = (lax.broadcasted_iota(jnp.int32, (D, D), 0)
           == lax.broadcasted_iota(jnp.int32, (D, D), 1)
           ).astype(jnp.float32)
    t = lax.dot_general(eye, w2, (((1,), (1,)), ((), ())),
                        precision=lax.Precision.HIGHEST,
                        preferred_element_type=jnp.float32)
    nl = pos_ref.shape[1]
    onehot = (lax.broadcasted_iota(jnp.int32, (nl, 1), 0)
              == l).astype(jnp.float32)
    pos_col = lax.dot_general(pos_ref[...], onehot, (((1,), (0,)), ((), ())),
                              precision=lax.Precision.HIGHEST,
                              preferred_element_type=jnp.float32)
    o_ref[0] = t + pos_col


@jax.jit
def _addpos(w3, pos_t):
    Lx, Bx, _ = w3.shape
    return pl.pallas_call(
        _addpos_body,
        grid=(Lx,),
        in_specs=[
            pl.BlockSpec((1, Bx, D), lambda l: (l, 0, 0)),
            pl.BlockSpec((D, Lx), lambda l: (0, 0)),
        ],
        out_specs=pl.BlockSpec((1, D, Bx), lambda l: (l, 0, 0)),
        out_shape=jax.ShapeDtypeStruct((Lx, D, Bx), jnp.float32),
    )(w3, pos_t)


def kernel(word_table, pos_table, x):
    Bx, Lx = x.shape
    n_rows = Bx * Lx
    xt2d = x.T.reshape(n_rows // SUB, SUB).astype(jnp.int32)
    w = _gather(word_table, xt2d, n_rows)
    out_t = _addpos(w.reshape(Lx, Bx, D), pos_table.T)
    return out_t.transpose(2, 0, 1)
